# baseline (speedup 1.0000x reference)
import dataclasses
import functools

import jax
import jax.numpy as jnp
from jax import lax
from jax.experimental import pallas as pl
from jax.experimental.pallas import tpu as pltpu
from jax.experimental.pallas import tpu_sc as plsc

N_NODES = 10000
N_EDGES = 160000
IN_F = 256
H_F = 512
FB = 128

NC = 2
NS = 16
EPS_SC = N_EDGES // NS
CHUNK = 80
N_CHUNKS = EPS_SC // CHUNK
RPS = 624
RPS_TAIL = N_NODES - NS * RPS


def _make_sc_segsum(n_feat_chunks: int, with_deg: bool):
    mesh = plsc.VectorSubcoreMesh(core_axis_name="c", subcore_axis_name="s")
    cpc = n_feat_chunks // NC

    out_type = [jax.ShapeDtypeStruct((n_feat_chunks, N_NODES, FB), jnp.float32)]
    if with_deg:
        out_type.append(jax.ShapeDtypeStruct((NS * N_NODES,), jnp.float32))

    nbuf = 2
    scratch_types = (
        [pltpu.VMEM((EPS_SC,), jnp.int32),
         pltpu.VMEM((EPS_SC,), jnp.int32)]
        + [pltpu.VMEM((CHUNK, FB), jnp.float32) for _ in range(nbuf)]
        + [pltpu.VMEM((CHUNK,), jnp.int32) for _ in range(nbuf)]
        + [pltpu.VMEM_SHARED((N_NODES, FB), jnp.float32)]
        + [pltpu.SemaphoreType.DMA for _ in range(nbuf)]
        + [pltpu.SemaphoreType.DMA for _ in range(nbuf)]
    )
    if with_deg:
        scratch_types.append(pltpu.VMEM((N_NODES,), jnp.float32))

    cp = pltpu.CompilerParams()
    if with_deg and "needs_layout_passes" in pltpu.CompilerParams.__dataclass_fields__:
        cp = dataclasses.replace(cp, needs_layout_passes=False)

    @functools.partial(pl.kernel, out_type=out_type, mesh=mesh,
                       scratch_types=scratch_types, compiler_params=cp)
    def seg_kernel(*refs):
        n_in = 4
        n_out = 2 if with_deg else 1
        x_hbm, src_hbm, dst_hbm, z_hbm = refs[:n_in]
        out_hbm = refs[n_in]
        deg_hbm = refs[n_in + 1] if with_deg else None
        sc = list(refs[n_in + n_out:])
        idx_s, idx_d = sc[0], sc[1]
        rows = sc[2:2 + nbuf]
        sdx = sc[2 + nbuf:2 + 2 * nbuf]
        acc = sc[2 + 2 * nbuf]
        gsem = sc[3 + 2 * nbuf:3 + 3 * nbuf]
        ssem = sc[3 + 3 * nbuf:3 + 4 * nbuf]
        degbuf = sc[3 + 4 * nbuf] if with_deg else None

        c = lax.axis_index("c")
        s = lax.axis_index("s")
        r0 = s * RPS
        e0 = s * EPS_SC
        tail0 = NS * RPS

        def stripe_copy(mk_src, mk_dst):
            pltpu.sync_copy(mk_src(r0, RPS), mk_dst(r0, RPS))

            @pl.when(s == NS - 1)
            def _():
                pltpu.sync_copy(mk_src(tail0, RPS_TAIL), mk_dst(tail0, RPS_TAIL))

        if with_deg:
            @pl.loop(0, N_NODES // 16)
            def _(i):
                degbuf[pl.ds(i * 16, 16)] = jnp.zeros((16,), jnp.float32)

        pltpu.sync_copy(src_hbm.at[pl.ds(e0, EPS_SC)], idx_s)
        pltpu.sync_copy(dst_hbm.at[pl.ds(e0, EPS_SC)], idx_d)

        def start_gather(slot, ci, fc):
            pltpu.async_copy(x_hbm.at[fc].at[idx_s.at[pl.ds(ci * CHUNK, CHUNK)]],
                             rows[slot], gsem[slot])

        def wait_gather(slot, fc):
            pltpu.make_async_copy(x_hbm.at[fc].at[pl.ds(0, CHUNK)],
                                  rows[slot], gsem[slot]).wait()

        def stage_idx(slot, ci, do_deg):
            @pl.loop(0, CHUNK // 16)
            def _(j):
                idxv = idx_d[pl.ds(ci * CHUNK + j * 16, 16)]
                sdx[slot][pl.ds(j * 16, 16)] = idxv
                if do_deg:
                    plsc.addupdate_scatter(degbuf, [idxv],
                                           jnp.full((16,), 1.0, jnp.float32))

        def start_scatter(slot):
            pltpu.async_copy(rows[slot], acc.at[sdx[slot]], ssem[slot],
                             add=True)

        def wait_scatter(slot):
            pltpu.make_async_copy(rows[slot], acc.at[sdx[slot]],
                                  ssem[slot]).wait()

        n_main = N_CHUNKS - (N_CHUNKS % nbuf or nbuf)
        n_rem = N_CHUNKS - n_main

        for cc in range(cpc):
            fc = c * cpc + cc
            stripe_copy(lambda o, n: z_hbm.at[pl.ds(o, n)],
                        lambda o, n: acc.at[pl.ds(o, n)])
            plsc.subcore_barrier()

            do_deg = with_deg and cc == 0

            for k in range(nbuf):
                start_gather(k, k, fc)

            @pl.loop(0, n_main, step=nbuf)
            def _(ci):
                for k in range(nbuf):
                    stage_idx(k, ci + k, do_deg)
                    wait_gather(k, fc)
                    start_scatter(k)
                for k in range(nbuf):
                    wait_scatter(k)
                    nxt = ci + k + nbuf

                    @pl.when(nxt < N_CHUNKS)
                    def _():
                        start_gather(k, nxt, fc)

            for j in range(n_rem):
                stage_idx(j, n_main + j, do_deg)
                wait_gather(j, fc)
                start_scatter(j)
            for j in range(n_rem):
                wait_scatter(j)

            plsc.subcore_barrier()
            stripe_copy(lambda o, n: acc.at[pl.ds(o, n)],
                        lambda o, n: out_hbm.at[fc].at[pl.ds(o, n)])
            if do_deg:
                @pl.when(c == 0)
                def _():
                    pltpu.sync_copy(degbuf, deg_hbm.at[pl.ds(s * N_NODES, N_NODES)])
            if cc + 1 < cpc:
                plsc.subcore_barrier()

    return seg_kernel


_sc_segsum_l1 = _make_sc_segsum(2, with_deg=True)
_sc_segsum_l2 = _make_sc_segsum(4, with_deg=False)

_BN = 1000


def _tc_layer1(x, s1p, deg, w1, b1):
    def body(x_ref, s_ref, deg_ref, w_ref, b_ref, out_ref):
        deg = jnp.sum(deg_ref[0], axis=0)[:, None]
        inv = 1.0 / jnp.maximum(deg, 1.0)
        agg = jnp.concatenate([s_ref[0], s_ref[1]], axis=-1) * inv
        h = jnp.dot((x_ref[...] + agg).astype(jnp.bfloat16),
                    w_ref[...].astype(jnp.bfloat16),
                    preferred_element_type=jnp.float32)
        h = jnp.maximum(h + b_ref[...], 0.0)
        for ch in range(4):
            out_ref[ch] = h[:, ch * FB:(ch + 1) * FB]

    return pl.pallas_call(
        body,
        grid=(N_NODES // _BN,),
        in_specs=[
            pl.BlockSpec((_BN, IN_F), lambda i: (i, 0)),
            pl.BlockSpec((2, _BN, FB), lambda i: (0, i, 0)),
            pl.BlockSpec((1, NS, _BN), lambda i: (i, 0, 0)),
            pl.BlockSpec((IN_F, H_F), lambda i: (0, 0)),
            pl.BlockSpec((1, H_F), lambda i: (0, 0)),
        ],
        out_specs=pl.BlockSpec((4, _BN, FB), lambda i: (0, i, 0)),
        out_shape=jax.ShapeDtypeStruct((4, N_NODES, FB), jnp.float32),
    )(x, s1p, deg, w1, b1.reshape(1, H_F))


def _tc_layer2(hp, s2p, deg, w2, b2):
    def body(h_ref, s_ref, deg_ref, w_ref, b_ref, out_ref):
        i = pl.program_id(0)
        deg = jnp.sum(deg_ref[0], axis=0)[:, None]
        inv = 1.0 / jnp.maximum(deg, 1.0)
        h = jnp.concatenate([h_ref[ch] for ch in range(4)], axis=-1)
        agg = jnp.concatenate([s_ref[ch] for ch in range(4)], axis=-1) * inv
        y = jnp.dot((h + agg).astype(jnp.bfloat16),
                    w_ref[...].astype(jnp.bfloat16),
                    preferred_element_type=jnp.float32)
        y = jnp.maximum(y + b_ref[...], 0.0)
        part = jnp.sum(y, axis=0, keepdims=True) * (1.0 / N_NODES)

        @pl.when(i == 0)
        def _():
            out_ref[...] = part

        @pl.when(i > 0)
        def _():
            out_ref[...] += part

    return pl.pallas_call(
        body,
        grid=(N_NODES // _BN,),
        in_specs=[
            pl.BlockSpec((4, _BN, FB), lambda i: (0, i, 0)),
            pl.BlockSpec((4, _BN, FB), lambda i: (0, i, 0)),
            pl.BlockSpec((1, NS, _BN), lambda i: (i, 0, 0)),
            pl.BlockSpec((H_F, H_F), lambda i: (0, 0)),
            pl.BlockSpec((1, H_F), lambda i: (0, 0)),
        ],
        out_specs=pl.BlockSpec((1, H_F), lambda i: (0, 0)),
        out_shape=jax.ShapeDtypeStruct((1, H_F), jnp.float32),
    )(hp, s2p, deg, w2, b2.reshape(1, H_F))


def kernel(in_feat, edge_index, W1, b1, W2, b2):
    src = edge_index[0].astype(jnp.int32)
    dst = edge_index[1].astype(jnp.int32)
    xp = in_feat.reshape(N_NODES, 2, FB).transpose(1, 0, 2)
    z = jnp.zeros((N_NODES, FB), jnp.float32)

    s1p, deg_p = _sc_segsum_l1(xp, src, dst, z)
    deg = deg_p.reshape(NS, N_NODES // _BN, _BN).transpose(1, 0, 2)
    hp = _tc_layer1(in_feat, s1p, deg, W1, b1)
    (s2p,) = _sc_segsum_l2(hp, src, dst, z)
    out = _tc_layer2(hp, s2p, deg, W2, b2)
    return out


# speedup vs baseline: 1.2456x; 1.2456x over previous
import dataclasses
import functools

import jax
import jax.numpy as jnp
from jax import lax
from jax.experimental import pallas as pl
from jax.experimental.pallas import tpu as pltpu
from jax.experimental.pallas import tpu_sc as plsc

N_NODES = 10000
N_EDGES = 160000
IN_F = 256
H_F = 512
FB = 128

NC = 2
NS = 16
EPS_SC = N_EDGES // NS
CHUNK = 80
N_CHUNKS = EPS_SC // CHUNK
RPS = 624
RPS_TAIL = N_NODES - NS * RPS


def _make_sc_segsum(n_feat_chunks: int, with_deg: bool):
    mesh = plsc.VectorSubcoreMesh(core_axis_name="c", subcore_axis_name="s")
    cpc = n_feat_chunks // NC

    out_type = [jax.ShapeDtypeStruct((n_feat_chunks, N_NODES, FB), jnp.float32)]
    if with_deg:
        out_type.append(jax.ShapeDtypeStruct((NS * N_NODES,), jnp.float32))

    nbuf = 2
    scratch_types = (
        [pltpu.VMEM((EPS_SC,), jnp.int32),
         pltpu.VMEM((EPS_SC,), jnp.int32)]
        + [pltpu.VMEM((CHUNK, FB), jnp.float32) for _ in range(nbuf)]
        + [pltpu.VMEM((CHUNK,), jnp.int32) for _ in range(nbuf)]
        + [pltpu.VMEM_SHARED((N_NODES, FB), jnp.float32)]
        + [pltpu.SemaphoreType.DMA for _ in range(nbuf)]
        + [pltpu.SemaphoreType.DMA for _ in range(nbuf)]
    )
    if with_deg:
        scratch_types.append(pltpu.VMEM((N_NODES,), jnp.float32))

    cp = pltpu.CompilerParams()
    if with_deg and "needs_layout_passes" in pltpu.CompilerParams.__dataclass_fields__:
        cp = dataclasses.replace(cp, needs_layout_passes=False)

    @functools.partial(pl.kernel, out_type=out_type, mesh=mesh,
                       scratch_types=scratch_types, compiler_params=cp)
    def seg_kernel(*refs):
        n_in = 4
        n_out = 2 if with_deg else 1
        x_hbm, src_hbm, dst_hbm, z_hbm = refs[:n_in]
        out_hbm = refs[n_in]
        deg_hbm = refs[n_in + 1] if with_deg else None
        sc = list(refs[n_in + n_out:])
        idx_s, idx_d = sc[0], sc[1]
        rows = sc[2:2 + nbuf]
        sdx = sc[2 + nbuf:2 + 2 * nbuf]
        acc = sc[2 + 2 * nbuf]
        gsem = sc[3 + 2 * nbuf:3 + 3 * nbuf]
        ssem = sc[3 + 3 * nbuf:3 + 4 * nbuf]
        degbuf = sc[3 + 4 * nbuf] if with_deg else None

        c = lax.axis_index("c")
        s = lax.axis_index("s")
        r0 = s * RPS
        e0 = s * EPS_SC
        tail0 = NS * RPS

        def stripe_copy(mk_src, mk_dst):
            pltpu.sync_copy(mk_src(r0, RPS), mk_dst(r0, RPS))

            @pl.when(s == NS - 1)
            def _():
                pltpu.sync_copy(mk_src(tail0, RPS_TAIL), mk_dst(tail0, RPS_TAIL))

        if with_deg:
            @pl.loop(0, N_NODES // 16)
            def _(i):
                degbuf[pl.ds(i * 16, 16)] = jnp.zeros((16,), jnp.float32)

        pltpu.sync_copy(src_hbm.at[pl.ds(e0, EPS_SC)], idx_s)
        pltpu.sync_copy(dst_hbm.at[pl.ds(e0, EPS_SC)], idx_d)

        def start_gather(slot, ci, fc):
            pltpu.async_copy(x_hbm.at[fc].at[idx_s.at[pl.ds(ci * CHUNK, CHUNK)]],
                             rows[slot], gsem[slot])

        def wait_gather(slot, fc):
            pltpu.make_async_copy(x_hbm.at[fc].at[pl.ds(0, CHUNK)],
                                  rows[slot], gsem[slot]).wait()

        def stage_idx(slot, ci, do_deg):
            @pl.loop(0, CHUNK // 16)
            def _(j):
                idxv = idx_d[pl.ds(ci * CHUNK + j * 16, 16)]
                sdx[slot][pl.ds(j * 16, 16)] = idxv
                if do_deg:
                    plsc.addupdate_scatter(degbuf, [idxv],
                                           jnp.full((16,), 1.0, jnp.float32))

        def scatter(slot):
            pltpu.sync_copy(rows[slot], acc.at[sdx[slot]], add=True)

        for cc in range(cpc):
            fc = c * cpc + cc
            stripe_copy(lambda o, n: z_hbm.at[pl.ds(o, n)],
                        lambda o, n: acc.at[pl.ds(o, n)])
            plsc.subcore_barrier()

            do_deg = with_deg and cc == 0

            start_gather(0, 0, fc)

            @pl.loop(0, N_CHUNKS - 1, step=2)
            def _(ci):
                start_gather(1, ci + 1, fc)
                stage_idx(0, ci, do_deg)
                wait_gather(0, fc)
                scatter(0)
                start_gather(0, ci + 2, fc)
                stage_idx(1, ci + 1, do_deg)
                wait_gather(1, fc)
                scatter(1)

            stage_idx(0, N_CHUNKS - 1, do_deg)
            wait_gather(0, fc)
            scatter(0)

            plsc.subcore_barrier()
            stripe_copy(lambda o, n: acc.at[pl.ds(o, n)],
                        lambda o, n: out_hbm.at[fc].at[pl.ds(o, n)])
            if do_deg:
                @pl.when(c == 0)
                def _():
                    pltpu.sync_copy(degbuf, deg_hbm.at[pl.ds(s * N_NODES, N_NODES)])
            if cc + 1 < cpc:
                plsc.subcore_barrier()

    return seg_kernel


_sc_segsum_l1 = _make_sc_segsum(2, with_deg=True)
_sc_segsum_l2 = _make_sc_segsum(4, with_deg=False)

_BN = 1000


def _tc_layer1(x, s1p, deg, w1, b1):
    def body(x_ref, s_ref, deg_ref, w_ref, b_ref, out_ref):
        deg = jnp.sum(deg_ref[0], axis=0)[:, None]
        inv = 1.0 / jnp.maximum(deg, 1.0)
        agg = jnp.concatenate([s_ref[0], s_ref[1]], axis=-1) * inv
        h = jnp.dot((x_ref[...] + agg).astype(jnp.bfloat16),
                    w_ref[...].astype(jnp.bfloat16),
                    preferred_element_type=jnp.float32)
        h = jnp.maximum(h + b_ref[...], 0.0)
        for ch in range(4):
            out_ref[ch] = h[:, ch * FB:(ch + 1) * FB]

    return pl.pallas_call(
        body,
        grid=(N_NODES // _BN,),
        in_specs=[
            pl.BlockSpec((_BN, IN_F), lambda i: (i, 0)),
            pl.BlockSpec((2, _BN, FB), lambda i: (0, i, 0)),
            pl.BlockSpec((1, NS, _BN), lambda i: (i, 0, 0)),
            pl.BlockSpec((IN_F, H_F), lambda i: (0, 0)),
            pl.BlockSpec((1, H_F), lambda i: (0, 0)),
        ],
        out_specs=pl.BlockSpec((4, _BN, FB), lambda i: (0, i, 0)),
        out_shape=jax.ShapeDtypeStruct((4, N_NODES, FB), jnp.float32),
    )(x, s1p, deg, w1, b1.reshape(1, H_F))


def _tc_layer2(hp, s2p, deg, w2, b2):
    def body(h_ref, s_ref, deg_ref, w_ref, b_ref, out_ref):
        i = pl.program_id(0)
        deg = jnp.sum(deg_ref[0], axis=0)[:, None]
        inv = 1.0 / jnp.maximum(deg, 1.0)
        h = jnp.concatenate([h_ref[ch] for ch in range(4)], axis=-1)
        agg = jnp.concatenate([s_ref[ch] for ch in range(4)], axis=-1) * inv
        y = jnp.dot((h + agg).astype(jnp.bfloat16),
                    w_ref[...].astype(jnp.bfloat16),
                    preferred_element_type=jnp.float32)
        y = jnp.maximum(y + b_ref[...], 0.0)
        part = jnp.sum(y, axis=0, keepdims=True) * (1.0 / N_NODES)

        @pl.when(i == 0)
        def _():
            out_ref[...] = part

        @pl.when(i > 0)
        def _():
            out_ref[...] += part

    return pl.pallas_call(
        body,
        grid=(N_NODES // _BN,),
        in_specs=[
            pl.BlockSpec((4, _BN, FB), lambda i: (0, i, 0)),
            pl.BlockSpec((4, _BN, FB), lambda i: (0, i, 0)),
            pl.BlockSpec((1, NS, _BN), lambda i: (i, 0, 0)),
            pl.BlockSpec((H_F, H_F), lambda i: (0, 0)),
            pl.BlockSpec((1, H_F), lambda i: (0, 0)),
        ],
        out_specs=pl.BlockSpec((1, H_F), lambda i: (0, 0)),
        out_shape=jax.ShapeDtypeStruct((1, H_F), jnp.float32),
    )(hp, s2p, deg, w2, b2.reshape(1, H_F))


def kernel(in_feat, edge_index, W1, b1, W2, b2):
    src = edge_index[0].astype(jnp.int32)
    dst = edge_index[1].astype(jnp.int32)
    xp = in_feat.reshape(N_NODES, 2, FB).transpose(1, 0, 2)
    z = jnp.zeros((N_NODES, FB), jnp.float32)

    s1p, deg_p = _sc_segsum_l1(xp, src, dst, z)
    deg = deg_p.reshape(NS, N_NODES // _BN, _BN).transpose(1, 0, 2)
    hp = _tc_layer1(in_feat, s1p, deg, W1, b1)
    (s2p,) = _sc_segsum_l2(hp, src, dst, z)
    out = _tc_layer2(hp, s2p, deg, W2, b2)
    return out
